# baseline (speedup 1.0000x reference)
import functools

import jax
import jax.numpy as jnp
from jax.experimental import pallas as pl
from jax.experimental.pallas import tpu as pltpu

B, A, IN, OUT, TOPK = 8, 2048, 128, 64, 16
SCALE = 8.0
BM = 256
NEG = -1e30


NSL = 16
SLW = A // NSL


def _gat_kernel(x_ref, conn_ref, wq_ref, wk_ref, wv_ref, gb_ref, out_ref,
                q_scr, k_scr, v_scr, cand_scr, segm_scr, t_scr):
    i = pl.program_id(1)

    @pl.when(i == 0)
    def _():
        xb = x_ref[0]
        q_scr[...] = jax.lax.dot_general(
            xb, wq_ref[...], (((1,), (1,)), ((), ())),
            preferred_element_type=jnp.float32)
        k_scr[...] = jax.lax.dot_general(
            xb, wk_ref[...], (((1,), (1,)), ((), ())),
            preferred_element_type=jnp.float32)
        v_scr[...] = jax.lax.dot_general(
            xb, wv_ref[...], (((1,), (1,)), ((), ())),
            preferred_element_type=jnp.float32)

    qb = q_scr[pl.ds(i * BM, BM), :]
    s = jax.lax.dot_general(
        qb, k_scr[...], (((1,), (1,)), ((), ())),
        preferred_element_type=jnp.float32)
    s = s * (1.0 / SCALE) + conn_ref[0]

    slices = [s[:, k * SLW:(k + 1) * SLW] for k in range(NSL)]

    def class_max(vals):
        m = vals[0]
        for v in vals[1:]:
            m = jnp.maximum(m, v)
        return m

    def peel(segm, j):
        cand_scr[j] = segm
        return class_max([jnp.where(sl < segm, sl, NEG) for sl in slices])

    def finish(k):
        def fbody(_, m):
            vals = [cand_scr[j] for j in range(k)]
            nm = class_max([jnp.where(c < m, c, NEG) for c in vals])
            return jnp.max(nm, axis=-1, keepdims=True)

        t_scr[...] = jax.lax.fori_loop(0, TOPK - 1, fbody, m1)

    segm = class_max(slices)
    m1 = jnp.max(segm, axis=-1, keepdims=True)

    for j in range(4):
        segm = peel(segm, j)
    segm_scr[...] = segm
    finish(4)
    r1 = jnp.max(segm, axis=-1, keepdims=True)
    done1 = jnp.all(t_scr[...] >= r1)

    @pl.when(jnp.logical_not(done1))
    def _():
        sg = segm_scr[...]
        for j in range(4, 6):
            sg = peel(sg, j)
        segm_scr[...] = sg
        finish(6)
        r2 = jnp.max(sg, axis=-1, keepdims=True)
        done2 = jnp.all(t_scr[...] >= r2)

        @pl.when(jnp.logical_not(done2))
        def _():
            sg2 = segm_scr[...]
            for j in range(6, TOPK):
                sg2 = peel(sg2, j)
            finish(TOPK)

    t = t_scr[...]

    w = jnp.where(s >= t, jnp.exp(s - m1), 0.0)
    z = jnp.sum(w, axis=-1, keepdims=True)
    o = jax.lax.dot_general(
        w, v_scr[...], (((1,), (0,)), ((), ())),
        preferred_element_type=jnp.float32)
    o = o / z

    mu = jnp.mean(o, axis=-1, keepdims=True)
    d = o - mu
    var = jnp.mean(d * d, axis=-1, keepdims=True)
    gamma = gb_ref[0:1, :]
    beta = gb_ref[1:2, :]
    out_ref[0] = d * jax.lax.rsqrt(var + 1e-5) * gamma + beta


@jax.jit
def kernel(x, connectivity, Wq, Wk, Wv, gamma, beta):
    gb = jnp.stack([gamma, beta], axis=0)
    grid = (B, A // BM)
    out = pl.pallas_call(
        _gat_kernel,
        grid=grid,
        in_specs=[
            pl.BlockSpec((1, A, IN), lambda b, i: (b, 0, 0)),
            pl.BlockSpec((1, BM, A), lambda b, i: (b, i, 0)),
            pl.BlockSpec((OUT, IN), lambda b, i: (0, 0)),
            pl.BlockSpec((OUT, IN), lambda b, i: (0, 0)),
            pl.BlockSpec((OUT, IN), lambda b, i: (0, 0)),
            pl.BlockSpec((2, OUT), lambda b, i: (0, 0)),
        ],
        out_specs=pl.BlockSpec((1, BM, OUT), lambda b, i: (b, i, 0)),
        out_shape=jax.ShapeDtypeStruct((B, A, OUT), jnp.float32),
        scratch_shapes=[
            pltpu.VMEM((A, OUT), jnp.float32),
            pltpu.VMEM((A, OUT), jnp.float32),
            pltpu.VMEM((A, OUT), jnp.float32),
            pltpu.VMEM((TOPK, BM, SLW), jnp.float32),
            pltpu.VMEM((BM, SLW), jnp.float32),
            pltpu.VMEM((BM, 1), jnp.float32),
        ],
        compiler_params=pltpu.CompilerParams(
            dimension_semantics=("arbitrary", "arbitrary"),
        ),
    )(x, connectivity, Wq, Wk, Wv, gb)
    return out


# speedup vs baseline: 1.1843x; 1.1843x over previous
import functools

import jax
import jax.numpy as jnp
from jax.experimental import pallas as pl
from jax.experimental.pallas import tpu as pltpu

B, A, IN, OUT, TOPK = 8, 2048, 128, 64, 16
SCALE = 8.0
BM = 512
NEG = -1e30


NSL = 16
SLW = A // NSL


def _gat_kernel(x_ref, conn_ref, wq_ref, wk_ref, wv_ref, gb_ref, out_ref,
                q_scr, k_scr, v_scr, cand_scr, segm_scr, t_scr):
    i = pl.program_id(1)

    @pl.when(i == 0)
    def _():
        xb = x_ref[0]
        q_scr[...] = jax.lax.dot_general(
            xb, wq_ref[...], (((1,), (1,)), ((), ())),
            preferred_element_type=jnp.float32)
        k_scr[...] = jax.lax.dot_general(
            xb, wk_ref[...], (((1,), (1,)), ((), ())),
            preferred_element_type=jnp.float32)
        v_scr[...] = jax.lax.dot_general(
            xb, wv_ref[...], (((1,), (1,)), ((), ())),
            preferred_element_type=jnp.float32)

    qb = q_scr[pl.ds(i * BM, BM), :]
    s = jax.lax.dot_general(
        qb, k_scr[...], (((1,), (1,)), ((), ())),
        preferred_element_type=jnp.float32)
    s = s * (1.0 / SCALE) + conn_ref[0]

    slices = [s[:, k * SLW:(k + 1) * SLW] for k in range(NSL)]

    def class_max(vals):
        m = vals[0]
        for v in vals[1:]:
            m = jnp.maximum(m, v)
        return m

    def peel(segm, j):
        cand_scr[j] = segm
        return class_max([jnp.where(sl < segm, sl, NEG) for sl in slices])

    def finish(k):
        def fbody(_, m):
            vals = [cand_scr[j] for j in range(k)]
            nm = class_max([jnp.where(c < m, c, NEG) for c in vals])
            return jnp.max(nm, axis=-1, keepdims=True)

        t_scr[...] = jax.lax.fori_loop(0, TOPK - 1, fbody, m1)

    segm = class_max(slices)
    m1 = jnp.max(segm, axis=-1, keepdims=True)

    for j in range(4):
        segm = peel(segm, j)
    segm_scr[...] = segm
    finish(4)
    r1 = jnp.max(segm, axis=-1, keepdims=True)
    done1 = jnp.all(t_scr[...] >= r1)

    @pl.when(jnp.logical_not(done1))
    def _():
        sg = segm_scr[...]
        for j in range(4, 6):
            sg = peel(sg, j)
        segm_scr[...] = sg
        finish(6)
        r2 = jnp.max(sg, axis=-1, keepdims=True)
        done2 = jnp.all(t_scr[...] >= r2)

        @pl.when(jnp.logical_not(done2))
        def _():
            sg2 = segm_scr[...]
            for j in range(6, TOPK):
                sg2 = peel(sg2, j)
            finish(TOPK)

    t = t_scr[...]

    w = jnp.where(s >= t, jnp.exp(s - m1), 0.0)
    z = jnp.sum(w, axis=-1, keepdims=True)
    o = jax.lax.dot_general(
        w, v_scr[...], (((1,), (0,)), ((), ())),
        preferred_element_type=jnp.float32)
    o = o / z

    mu = jnp.mean(o, axis=-1, keepdims=True)
    d = o - mu
    var = jnp.mean(d * d, axis=-1, keepdims=True)
    gamma = gb_ref[0:1, :]
    beta = gb_ref[1:2, :]
    out_ref[0] = d * jax.lax.rsqrt(var + 1e-5) * gamma + beta


@jax.jit
def kernel(x, connectivity, Wq, Wk, Wv, gamma, beta):
    gb = jnp.stack([gamma, beta], axis=0)
    grid = (B, A // BM)
    out = pl.pallas_call(
        _gat_kernel,
        grid=grid,
        in_specs=[
            pl.BlockSpec((1, A, IN), lambda b, i: (b, 0, 0)),
            pl.BlockSpec((1, BM, A), lambda b, i: (b, i, 0)),
            pl.BlockSpec((OUT, IN), lambda b, i: (0, 0)),
            pl.BlockSpec((OUT, IN), lambda b, i: (0, 0)),
            pl.BlockSpec((OUT, IN), lambda b, i: (0, 0)),
            pl.BlockSpec((2, OUT), lambda b, i: (0, 0)),
        ],
        out_specs=pl.BlockSpec((1, BM, OUT), lambda b, i: (b, i, 0)),
        out_shape=jax.ShapeDtypeStruct((B, A, OUT), jnp.float32),
        scratch_shapes=[
            pltpu.VMEM((A, OUT), jnp.float32),
            pltpu.VMEM((A, OUT), jnp.float32),
            pltpu.VMEM((A, OUT), jnp.float32),
            pltpu.VMEM((TOPK, BM, SLW), jnp.float32),
            pltpu.VMEM((BM, SLW), jnp.float32),
            pltpu.VMEM((BM, 1), jnp.float32),
        ],
        compiler_params=pltpu.CompilerParams(
            dimension_semantics=("arbitrary", "arbitrary"),
        ),
    )(x, connectivity, Wq, Wk, Wv, gb)
    return out


# speedup vs baseline: 1.4559x; 1.2293x over previous
import functools

import jax
import jax.numpy as jnp
from jax.experimental import pallas as pl
from jax.experimental.pallas import tpu as pltpu

B, A, IN, OUT, TOPK = 8, 2048, 128, 64, 16
SCALE = 8.0
BM = 512
NEG = -1e30


NSL = 16
SLW = A // NSL


def _gat_kernel(x_ref, conn_ref, wq_ref, wk_ref, wv_ref, gb_ref, out_ref,
                q_scr, k_scr, v_scr, cand_scr, segm_scr, t_scr):
    i = pl.program_id(1)

    @pl.when(i == 0)
    def _():
        xb = x_ref[0]
        q_scr[...] = jax.lax.dot_general(
            xb, wq_ref[...], (((1,), (1,)), ((), ())),
            preferred_element_type=jnp.float32)
        k_scr[...] = jax.lax.dot_general(
            xb, wk_ref[...], (((1,), (1,)), ((), ())),
            preferred_element_type=jnp.float32)
        v_scr[...] = jax.lax.dot_general(
            xb, wv_ref[...], (((1,), (1,)), ((), ())),
            preferred_element_type=jnp.float32)

    qb = q_scr[pl.ds(i * BM, BM), :]
    s = jax.lax.dot_general(
        qb, k_scr[...], (((1,), (1,)), ((), ())),
        preferred_element_type=jnp.float32)
    s = s * (1.0 / SCALE) + conn_ref[0]

    slices = [s[:, k * SLW:(k + 1) * SLW] for k in range(NSL)]

    def class_max(vals):
        m = vals[0]
        for v in vals[1:]:
            m = jnp.maximum(m, v)
        return m

    def peel(segm, j):
        cand_scr[j] = segm
        return class_max([jnp.where(sl < segm, sl, NEG) for sl in slices])

    def finish(k):
        vals = [cand_scr[j] for j in range(k)]
        hb = BM // 2
        halves = []
        for off in (0, hb):
            hv = [v[off:off + hb] for v in vals]
            m = m1[off:off + hb]
            for _ in range(TOPK - 1):
                nm = class_max([jnp.where(c < m, c, NEG) for c in hv])
                m = jnp.max(nm, axis=-1, keepdims=True)
            halves.append(m)
        t_scr[...] = jnp.concatenate(halves, axis=0)

    segm = class_max(slices)
    m1 = jnp.max(segm, axis=-1, keepdims=True)

    for j in range(4):
        segm = peel(segm, j)
    segm_scr[...] = segm
    finish(4)
    r1 = jnp.max(segm, axis=-1, keepdims=True)
    done1 = jnp.all(t_scr[...] >= r1)

    @pl.when(jnp.logical_not(done1))
    def _():
        sg = segm_scr[...]
        for j in range(4, 6):
            sg = peel(sg, j)
        segm_scr[...] = sg
        finish(6)
        r2 = jnp.max(sg, axis=-1, keepdims=True)
        done2 = jnp.all(t_scr[...] >= r2)

        @pl.when(jnp.logical_not(done2))
        def _():
            sg2 = segm_scr[...]
            for j in range(6, TOPK):
                sg2 = peel(sg2, j)
            finish(TOPK)

    t = t_scr[...]

    w = jnp.where(s >= t, jnp.exp(s - m1), 0.0)
    z = jnp.sum(w, axis=-1, keepdims=True)
    o = jax.lax.dot_general(
        w, v_scr[...], (((1,), (0,)), ((), ())),
        preferred_element_type=jnp.float32)
    o = o / z

    mu = jnp.mean(o, axis=-1, keepdims=True)
    d = o - mu
    var = jnp.mean(d * d, axis=-1, keepdims=True)
    gamma = gb_ref[0:1, :]
    beta = gb_ref[1:2, :]
    out_ref[0] = d * jax.lax.rsqrt(var + 1e-5) * gamma + beta


@jax.jit
def kernel(x, connectivity, Wq, Wk, Wv, gamma, beta):
    gb = jnp.stack([gamma, beta], axis=0)
    grid = (B, A // BM)
    out = pl.pallas_call(
        _gat_kernel,
        grid=grid,
        in_specs=[
            pl.BlockSpec((1, A, IN), lambda b, i: (b, 0, 0)),
            pl.BlockSpec((1, BM, A), lambda b, i: (b, i, 0)),
            pl.BlockSpec((OUT, IN), lambda b, i: (0, 0)),
            pl.BlockSpec((OUT, IN), lambda b, i: (0, 0)),
            pl.BlockSpec((OUT, IN), lambda b, i: (0, 0)),
            pl.BlockSpec((2, OUT), lambda b, i: (0, 0)),
        ],
        out_specs=pl.BlockSpec((1, BM, OUT), lambda b, i: (b, i, 0)),
        out_shape=jax.ShapeDtypeStruct((B, A, OUT), jnp.float32),
        scratch_shapes=[
            pltpu.VMEM((A, OUT), jnp.float32),
            pltpu.VMEM((A, OUT), jnp.float32),
            pltpu.VMEM((A, OUT), jnp.float32),
            pltpu.VMEM((TOPK, BM, SLW), jnp.float32),
            pltpu.VMEM((BM, SLW), jnp.float32),
            pltpu.VMEM((BM, 1), jnp.float32),
        ],
        compiler_params=pltpu.CompilerParams(
            dimension_semantics=("arbitrary", "arbitrary"),
        ),
    )(x, connectivity, Wq, Wk, Wv, gb)
    return out
